# baseline (speedup 1.0000x reference)
import functools

import jax
import jax.numpy as jnp
from jax.experimental import pallas as pl

_HIDDEN = 4096
_EXPERTS = 64
_TOPK = 8


_SUB = 128


def _router_body(x_ref, w_ref, logits_ref, wts_ref, idx_ref):
    b = x_ref.shape[0]
    e = _EXPERTS
    lane_f = jax.lax.broadcasted_iota(jnp.int32, (_SUB, e), 1).astype(jnp.float32)
    kcol = jax.lax.broadcasted_iota(jnp.int32, (_SUB, _TOPK), 1)
    for s in range(b // _SUB):
        rows = pl.ds(s * _SUB, _SUB)
        logits = jax.lax.dot_general(
            x_ref[rows, :], w_ref[...],
            dimension_numbers=(((1,), (1,)), ((), ())),
            preferred_element_type=jnp.float32,
        )
        logits_ref[rows, :] = logits
        work = logits
        vals = jnp.zeros((_SUB, _TOPK), jnp.float32)
        ms = []
        for j in range(_TOPK):
            m = jnp.max(work, axis=1, keepdims=True)
            ms.append(m)
            vals = jnp.where(kcol == j, m, vals)
            work = jnp.where(work == m, -jnp.inf, work)
        lg = logits_ref[rows, :]
        idxs_f = jnp.zeros((_SUB, _TOPK), jnp.float32)
        for j in range(_TOPK):
            imf = jnp.min(jnp.where(lg == ms[j], lane_f, float(e)),
                          axis=1, keepdims=True)
            idxs_f = jnp.where(kcol == j, imf, idxs_f)
        ex = jnp.exp(vals - jnp.max(vals, axis=1, keepdims=True))
        wts_ref[rows, :] = ex / jnp.sum(ex, axis=1, keepdims=True)
        idx_ref[rows, :] = idxs_f.astype(jnp.int32)


@functools.partial(jax.jit, static_argnames=("block_t", "interpret"))
def _router(hidden_states, gate_w, block_t=1024, interpret=False):
    tokens = hidden_states.shape[0]
    grid = (tokens // block_t,)
    return pl.pallas_call(
        _router_body,
        grid=grid,
        in_specs=[
            pl.BlockSpec((block_t, _HIDDEN), lambda i: (i, 0)),
            pl.BlockSpec((_EXPERTS, _HIDDEN), lambda i: (0, 0)),
        ],
        out_specs=[
            pl.BlockSpec((block_t, _EXPERTS), lambda i: (i, 0)),
            pl.BlockSpec((block_t, _TOPK), lambda i: (i, 0)),
            pl.BlockSpec((block_t, _TOPK), lambda i: (i, 0)),
        ],
        out_shape=[
            jax.ShapeDtypeStruct((tokens, _EXPERTS), jnp.float32),
            jax.ShapeDtypeStruct((tokens, _TOPK), jnp.float32),
            jax.ShapeDtypeStruct((tokens, _TOPK), jnp.int32),
        ],
        interpret=interpret,
    )(hidden_states, gate_w)


def kernel(hidden_states, gate_w):
    logits, wts, idxs = _router(hidden_states, gate_w)
    return (wts, idxs, logits)
